# baseline (speedup 1.0000x reference)
import jax
import jax.numpy as jnp
from jax import lax
from jax.experimental import pallas as pl
from jax.experimental.pallas import tpu as pltpu

_B = 16
_D = 64
_HW = 1024
_N_TOK = _B * _HW


_N_CHUNK = 8


def _fill_body(emb_ref, q_hbm, enc_hbm, q_v, enc_v, sem_q, sem_e):
    col = emb_ref[...]
    q_v[...] = lax.broadcast_in_dim(col, (_B, _D, _HW), (1, 2))
    enc_v[...] = jnp.full((128, 128), 1.0, jnp.float32)
    step = _B // _N_CHUNK
    copies = [
        pltpu.make_async_copy(
            q_v.at[pl.ds(i * step, step)],
            q_hbm.at[pl.ds(i * step, step)],
            sem_q.at[i],
        )
        for i in range(_N_CHUNK)
    ]
    ce = pltpu.make_async_copy(enc_v, enc_hbm, sem_e)
    for c in copies:
        c.start()
    ce.start()
    for c in copies:
        c.wait()
    ce.wait()


def kernel(x, embeddings):
    del x
    emb_col = embeddings.reshape(_D, 1)
    q3, enc2 = pl.pallas_call(
        _fill_body,
        in_specs=[pl.BlockSpec(memory_space=pltpu.VMEM)],
        out_specs=[
            pl.BlockSpec(memory_space=pl.ANY),
            pl.BlockSpec(memory_space=pl.ANY),
        ],
        out_shape=[
            jax.ShapeDtypeStruct((_B, _D, _HW), jnp.float32),
            jax.ShapeDtypeStruct((128, 128), jnp.float32),
        ],
        scratch_shapes=[
            pltpu.VMEM((_B, _D, _HW), jnp.float32),
            pltpu.VMEM((128, 128), jnp.float32),
            pltpu.SemaphoreType.DMA((_N_CHUNK,)),
            pltpu.SemaphoreType.DMA,
        ],
    )(emb_col)
    quantized = q3.reshape(_B, _D, 32, 32)
    encodings = enc2.reshape(_N_TOK, 1)
    return (encodings, quantized)
